# baseline (speedup 1.0000x reference)
import jax
import jax.numpy as jnp
from jax import lax
from jax.experimental import pallas as pl
from jax.experimental.pallas import tpu as pltpu
from jax.experimental.pallas import tpu_sc as plsc

N = 10000
D = 128
DQ = D // 4
NQ = 4
E = 320000
NC = 2
NS = 16
NW = NC * NS
BATCH = 128
NB_DEG = 80
NB = 160
T_EDGES = NB * BATCH
E_PAD = NS * T_EDGES
NROWS = N
ROWS_PER_TILE = NROWS // NS
RFULL = ROWS_PER_TILE // BATCH
RTAIL = ROWS_PER_TILE - RFULL * BATCH


def _agg_body(g, srcr, dstr, ewf, flagr, out, srcb, dstb, ewb, flagb,
              rows0, rows1, rows2, rows3,
              gs0, gs1, gs2, gs3, ss0, ss1, ss2, ss3, acc):
    c = lax.axis_index("c")
    s = lax.axis_index("s")
    rows = [rows0, rows1, rows2, rows3]
    gsem = [gs0, gs1, gs2, gs3]
    ssem = [ss0, ss1, ss2, ss3]

    pltpu.sync_copy(srcr.at[s], srcb)
    pltpu.sync_copy(dstr.at[s], dstb)
    pltpu.sync_copy(ewf.at[s, 0], ewb)
    pltpu.sync_copy(flagr, flagb)
    one_pass = flagb[...][0]

    def z_body(j, carry):
        for k in range(DQ // 16):
            rows0[j, pl.ds(k * 16, 16)] = jnp.zeros((16,), jnp.float32)
        return carry

    base = s * ROWS_PER_TILE

    def fire_gather(q, slot, b):
        pltpu.async_copy(g.at[q].at[srcb.at[b]], rows[slot], gsem[slot])

    def wait_gather(slot):
        pltpu.make_async_copy(g.at[0].at[pl.ds(0, BATCH)], rows[slot],
                              gsem[slot]).wait()

    def wait_scatter(slot):
        pltpu.make_async_copy(g.at[0].at[pl.ds(0, BATCH)], rows[slot],
                              ssem[slot]).wait()

    def run_pass(q):
        lax.fori_loop(0, BATCH, z_body, None)
        for i in range(RFULL):
            pltpu.sync_copy(rows0, acc.at[pl.ds(base + i * BATCH, BATCH)])
        pltpu.sync_copy(rows0.at[pl.ds(0, RTAIL)],
                        acc.at[pl.ds(base + RFULL * BATCH, RTAIL)])
        plsc.subcore_barrier()

        fire_gather(q, 0, 0)
        fire_gather(q, 1, 1)
        fire_gather(q, 2, 2)

        def grp(t, carry):
            for r in range(4):
                b = t * 4 + r
                wait_gather(r)
                eb = b * BATCH

                def sc_body(qq, carry2):
                    chunk = ewb[pl.ds(eb + qq * 16, 16)]
                    for t2 in range(16):
                        ev = chunk[t2]
                        j = qq * 16 + t2
                        for k in range(DQ // 16):
                            sl = pl.ds(k * 16, 16)
                            rows[r][j, sl] = rows[r][j, sl] * ev
                    return carry2

                lax.fori_loop(0, BATCH // 16, sc_body, None)
                pltpu.async_copy(rows[r], acc.at[dstb.at[b]], ssem[r],
                                 add=True)

                slot_n = (r + 3) % 4

                @pl.when(b + 3 < NB)
                def _():
                    @pl.when(b >= 1)
                    def _():
                        wait_scatter(slot_n)
                    fire_gather(q, slot_n, b + 3)
            return carry

        lax.fori_loop(0, NB // 4, grp, None)
        for r in range(4):
            wait_scatter(r)
        plsc.subcore_barrier()
        sl = pl.ds(base, ROWS_PER_TILE)
        pltpu.sync_copy(acc.at[sl], out.at[q, sl])
        plsc.subcore_barrier()

    def run_deg():
        def zr_body(j, carry):
            for rr in range(4):
                for k in range(DQ // 16):
                    rows[rr][j, pl.ds(k * 16, 16)] = jnp.zeros((16,),
                                                              jnp.float32)
            return carry

        lax.fori_loop(0, BATCH, zr_body, None)
        for i in range(RFULL):
            pltpu.sync_copy(rows0, acc.at[pl.ds(base + i * BATCH, BATCH)])
        pltpu.sync_copy(rows0.at[pl.ds(0, RTAIL)],
                        acc.at[pl.ds(base + RFULL * BATCH, RTAIL)])
        plsc.subcore_barrier()

        def dgrp(t, carry):
            for r in range(4):
                b = t * 4 + r
                eb = b * BATCH

                @pl.when(b >= 4)
                def _():
                    wait_scatter(r)

                def dfill(qq, carry2):
                    chunk = ewb[pl.ds(eb + qq * 16, 16)]
                    for t2 in range(16):
                        j = qq * 16 + t2
                        rows[r][j, pl.ds(0, 16)] = jnp.full(
                            (16,), chunk[t2], jnp.float32)
                    return carry2

                lax.fori_loop(0, BATCH // 16, dfill, None)
                pltpu.async_copy(rows[r], acc.at[dstb.at[b]], ssem[r],
                                 add=True)
            return carry

        lax.fori_loop(0, NB // 4, dgrp, None)
        for r in range(4):
            wait_scatter(r)
        plsc.subcore_barrier()
        sl = pl.ds(base, ROWS_PER_TILE)
        pltpu.sync_copy(acc.at[sl], out.at[0, sl])

    @pl.when(one_pass == 0)
    def _():
        run_pass(c)
        run_pass(2 + c)

    @pl.when(one_pass == 1)
    def _():
        run_deg()


_agg_call = pl.kernel(
    _agg_body,
    out_type=jax.ShapeDtypeStruct((NQ, NROWS, DQ), jnp.float32),
    compiler_params=pltpu.CompilerParams(use_tc_tiling_on_sc=False),
    mesh=plsc.VectorSubcoreMesh(core_axis_name="c", subcore_axis_name="s"),
    scratch_types=[
        pltpu.VMEM((NB, BATCH), jnp.int32),
        pltpu.VMEM((NB, BATCH), jnp.int32),
        pltpu.VMEM((T_EDGES,), jnp.float32),
        pltpu.VMEM((16,), jnp.int32),
        pltpu.VMEM((BATCH, DQ), jnp.float32),
        pltpu.VMEM((BATCH, DQ), jnp.float32),
        pltpu.VMEM((BATCH, DQ), jnp.float32),
        pltpu.VMEM((BATCH, DQ), jnp.float32),
        pltpu.SemaphoreType.DMA,
        pltpu.SemaphoreType.DMA,
        pltpu.SemaphoreType.DMA,
        pltpu.SemaphoreType.DMA,
        pltpu.SemaphoreType.DMA,
        pltpu.SemaphoreType.DMA,
        pltpu.SemaphoreType.DMA,
        pltpu.SemaphoreType.DMA,
        pltpu.VMEM_SHARED((NROWS, DQ), jnp.float32),
    ],
)


_BLK = 1000
_GRID = N // _BLK

_spec_col = pl.BlockSpec((_BLK, 1), lambda i: (i, 0))
_spec_row = pl.BlockSpec((_BLK, D), lambda i: (i, 0))
_spec_half = pl.BlockSpec((NQ, _BLK, DQ), lambda i: (0, i, 0))
_spec_w = pl.BlockSpec((D, D), lambda i: (0, 0))
_spec_b = pl.BlockSpec((1, D), lambda i: (0, 0))


def _step_body(if0, a, g, dis, x, bt, wt, y_out, gn_out, dis_out):
    is0 = if0[0, 0] > 0.0
    dv = dis[...]
    y = jnp.concatenate([dv * (a[q] + g[q]) for q in range(NQ)],
                        axis=1) + bt[...]
    y_out[...] = y
    z = jnp.maximum(y, 0.0)
    disn = jnp.where(is0, lax.rsqrt(a[0][:, 0:1] + 1.0), dv)
    zin = jnp.where(is0, x[...], z)
    h = jnp.dot(zin, wt[...], preferred_element_type=jnp.float32) * disn
    for q in range(NQ):
        gn_out[q] = h[:, q * DQ:(q + 1) * DQ]
    dis_out[...] = disn


def _step_call(if0, a, g, dis, x, bt, wt):
    return pl.pallas_call(
        _step_body,
        grid=(_GRID,),
        in_specs=[pl.BlockSpec((1, 1), lambda i: (0, 0)),
                  _spec_half, _spec_half, _spec_col, _spec_row,
                  _spec_b, _spec_w],
        out_specs=[_spec_row, _spec_half, _spec_col],
        out_shape=[
            jax.ShapeDtypeStruct((N, D), jnp.float32),
            jax.ShapeDtypeStruct((NQ, N, DQ), jnp.float32),
            jax.ShapeDtypeStruct((N, 1), jnp.float32),
        ],
    )(if0, a, g, dis, x, bt, wt)


def kernel(x, edge_index, edge_weight, W1, b1, W2, b2):
    src = edge_index[0].astype(jnp.int32)
    dst = edge_index[1].astype(jnp.int32)
    ew = edge_weight.astype(jnp.float32)
    pad = E_PAD - E
    pidx = jnp.arange(pad, dtype=jnp.int32) % N
    srcp = jnp.concatenate([src, pidx])
    dstp = jnp.concatenate([dst, pidx])
    ewp = jnp.concatenate([ew, jnp.zeros((pad,), jnp.float32)])
    srcr = srcp.reshape(NS, NB, BATCH)
    dstr = dstp.reshape(NS, NB, BATCH)
    ewf = ewp.reshape(NS, 1, T_EDGES)

    n_steps = 3 + jnp.minimum(jnp.min(edge_weight[:8]), 0.0).astype(jnp.int32)
    bs = jnp.stack([b1.reshape(1, D), b1.reshape(1, D), b2.reshape(1, D)])
    ws = jnp.stack([W1, W2, W2])

    def cond(state):
        i, _, _, _ = state
        return i < n_steps

    def step(state):
        i, g, dis, _ = state
        is0 = (i == 0).astype(jnp.int32)
        flag = jnp.full((16,), is0)
        a = _agg_call(g, srcr, dstr, ewf, flag)
        y, g_next, dis_next = _step_call(
            is0.astype(jnp.float32).reshape(1, 1), a, g, dis, x,
            lax.dynamic_index_in_dim(bs, i, keepdims=False),
            lax.dynamic_index_in_dim(ws, i, keepdims=False))
        return i + 1, g_next, dis_next, y

    g_ones = jnp.ones((NQ, N, DQ), jnp.float32)
    _, _, _, out = lax.while_loop(
        cond, step, (jnp.int32(0), g_ones, jnp.zeros((N, 1), jnp.float32), x))
    return out


# speedup vs baseline: 1.0332x; 1.0332x over previous
import jax
import jax.numpy as jnp
from jax import lax
from jax.experimental import pallas as pl
from jax.experimental.pallas import tpu as pltpu
from jax.experimental.pallas import tpu_sc as plsc

N = 10000
D = 128
DQ = D // 4
NQ = 4
E = 320000
NC = 2
NS = 16
NW = NC * NS
BATCH = 128
NB_DEG = 80
NB = 160
T_EDGES = NB * BATCH
E_PAD = NS * T_EDGES
NROWS = N
ROWS_PER_TILE = NROWS // NS
RFULL = ROWS_PER_TILE // BATCH
RTAIL = ROWS_PER_TILE - RFULL * BATCH


def _agg_body(g, srcr, dstr, ewf, flagr, out, srcb, dstb, ewb, flagb,
              rows0, rows1, rows2, rows3,
              gs0, gs1, gs2, gs3, ss0, ss1, ss2, ss3, acc):
    c = lax.axis_index("c")
    s = lax.axis_index("s")
    rows = [rows0, rows1, rows2, rows3]
    gsem = [gs0, gs1, gs2, gs3]
    ssem = [ss0, ss1, ss2, ss3]

    pltpu.sync_copy(srcr.at[s], srcb)
    pltpu.sync_copy(dstr.at[s], dstb)
    pltpu.sync_copy(ewf.at[s, 0], ewb)
    pltpu.sync_copy(flagr, flagb)
    one_pass = flagb[...][0]

    def z_body(j, carry):
        for k in range(DQ // 16):
            rows0[j, pl.ds(k * 16, 16)] = jnp.zeros((16,), jnp.float32)
        return carry

    base = s * ROWS_PER_TILE

    def fire_gather(q, slot, b):
        pltpu.async_copy(g.at[q].at[srcb.at[b]], rows[slot], gsem[slot])

    def wait_gather(slot):
        pltpu.make_async_copy(g.at[0].at[pl.ds(0, BATCH)], rows[slot],
                              gsem[slot]).wait()

    def wait_scatter(slot):
        pltpu.make_async_copy(g.at[0].at[pl.ds(0, BATCH)], rows[slot],
                              ssem[slot]).wait()

    def run_pass(q):
        lax.fori_loop(0, BATCH, z_body, None)
        for i in range(RFULL):
            pltpu.sync_copy(rows0, acc.at[pl.ds(base + i * BATCH, BATCH)])
        pltpu.sync_copy(rows0.at[pl.ds(0, RTAIL)],
                        acc.at[pl.ds(base + RFULL * BATCH, RTAIL)])
        plsc.subcore_barrier()

        fire_gather(q, 0, 0)
        fire_gather(q, 1, 1)
        fire_gather(q, 2, 2)

        def grp(t, carry):
            for r in range(4):
                b = t * 4 + r
                wait_gather(r)
                eb = b * BATCH

                def sc_body(qq, carry2):
                    chunk = ewb[pl.ds(eb + qq * 16, 16)]
                    for t2 in range(16):
                        ev = chunk[t2]
                        j = qq * 16 + t2
                        for k in range(DQ // 16):
                            sl = pl.ds(k * 16, 16)
                            rows[r][j, sl] = rows[r][j, sl] * ev
                    return carry2

                lax.fori_loop(0, BATCH // 16, sc_body, None)
                pltpu.async_copy(rows[r], acc.at[dstb.at[b]], ssem[r],
                                 add=True)

                slot_n = (r + 3) % 4

                @pl.when(b + 3 < NB)
                def _():
                    @pl.when(b >= 1)
                    def _():
                        wait_scatter(slot_n)
                    fire_gather(q, slot_n, b + 3)
            return carry

        lax.fori_loop(0, NB // 4, grp, None)
        for r in range(4):
            wait_scatter(r)
        plsc.subcore_barrier()
        sl = pl.ds(base, ROWS_PER_TILE)
        pltpu.sync_copy(acc.at[sl], out.at[q, sl])
        plsc.subcore_barrier()

    def run_deg():
        def zr_body(j, carry):
            for rr in range(4):
                for k in range(DQ // 16):
                    rows[rr][j, pl.ds(k * 16, 16)] = jnp.zeros((16,),
                                                              jnp.float32)
            return carry

        lax.fori_loop(0, BATCH, zr_body, None)
        for i in range(RFULL):
            pltpu.sync_copy(rows0, acc.at[pl.ds(base + i * BATCH, BATCH)])
        pltpu.sync_copy(rows0.at[pl.ds(0, RTAIL)],
                        acc.at[pl.ds(base + RFULL * BATCH, RTAIL)])
        plsc.subcore_barrier()

        def dgrp(t, carry):
            for r in range(4):
                b = t * 4 + r
                eb = b * BATCH

                @pl.when(b >= 4)
                def _():
                    wait_scatter(r)

                def dfill(qq, carry2):
                    chunk = ewb[pl.ds(eb + qq * 16, 16)]
                    for t2 in range(16):
                        j = qq * 16 + t2
                        rows[r][j, pl.ds(0, 16)] = jnp.full(
                            (16,), chunk[t2], jnp.float32)
                    return carry2

                lax.fori_loop(0, BATCH // 16, dfill, None)
                pltpu.async_copy(rows[r], acc.at[dstb.at[b]], ssem[r],
                                 add=True)
            return carry

        lax.fori_loop(0, NB // 4, dgrp, None)
        for r in range(4):
            wait_scatter(r)
        plsc.subcore_barrier()
        sl = pl.ds(base, ROWS_PER_TILE)
        pltpu.sync_copy(acc.at[sl], out.at[0, sl])

    @pl.when(one_pass == 0)
    def _():
        run_pass(c)
        run_pass(2 + c)

    @pl.when(one_pass == 1)
    def _():
        run_deg()


_agg_call = pl.kernel(
    _agg_body,
    out_type=jax.ShapeDtypeStruct((NQ, NROWS, DQ), jnp.float32),
    compiler_params=pltpu.CompilerParams(use_tc_tiling_on_sc=False),
    mesh=plsc.VectorSubcoreMesh(core_axis_name="c", subcore_axis_name="s"),
    scratch_types=[
        pltpu.VMEM((NB, BATCH), jnp.int32),
        pltpu.VMEM((NB, BATCH), jnp.int32),
        pltpu.VMEM((T_EDGES,), jnp.float32),
        pltpu.VMEM((16,), jnp.int32),
        pltpu.VMEM((BATCH, DQ), jnp.float32),
        pltpu.VMEM((BATCH, DQ), jnp.float32),
        pltpu.VMEM((BATCH, DQ), jnp.float32),
        pltpu.VMEM((BATCH, DQ), jnp.float32),
        pltpu.SemaphoreType.DMA,
        pltpu.SemaphoreType.DMA,
        pltpu.SemaphoreType.DMA,
        pltpu.SemaphoreType.DMA,
        pltpu.SemaphoreType.DMA,
        pltpu.SemaphoreType.DMA,
        pltpu.SemaphoreType.DMA,
        pltpu.SemaphoreType.DMA,
        pltpu.VMEM_SHARED((NROWS, DQ), jnp.float32),
    ],
)


_BLK = 1000
_GRID = N // _BLK

_spec_col = pl.BlockSpec((_BLK, 1), lambda i: (i, 0))
_spec_row = pl.BlockSpec((_BLK, D), lambda i: (i, 0))
_spec_half = pl.BlockSpec((NQ, _BLK, DQ), lambda i: (0, i, 0))
_spec_w = pl.BlockSpec((D, D), lambda i: (0, 0))
_spec_b = pl.BlockSpec((1, D), lambda i: (0, 0))


def _lin1_body(d0, x, w1, g_out, dis_out):
    dis = lax.rsqrt(d0[...] + 1.0)
    h = jnp.dot(x[...], w1[...], preferred_element_type=jnp.float32) * dis
    for q in range(NQ):
        g_out[q] = h[:, q * DQ:(q + 1) * DQ]
    dis_out[...] = dis


def _lin1(d0, x, w1):
    return pl.pallas_call(
        _lin1_body,
        grid=(_GRID,),
        in_specs=[_spec_col, _spec_row, _spec_w],
        out_specs=[_spec_half, _spec_col],
        out_shape=[
            jax.ShapeDtypeStruct((NQ, N, DQ), jnp.float32),
            jax.ShapeDtypeStruct((N, 1), jnp.float32),
        ],
    )(d0, x, w1)


def _layer_body(p, g, dis, bt, wt, y_out, gn_out):
    dv = dis[...]
    y = jnp.concatenate([dv * (p[q] + g[q]) for q in range(NQ)],
                        axis=1) + bt[...]
    y_out[...] = y
    z = jnp.maximum(y, 0.0)
    h = jnp.dot(z, wt[...], preferred_element_type=jnp.float32) * dv
    for q in range(NQ):
        gn_out[q] = h[:, q * DQ:(q + 1) * DQ]


def _layer(p, g, dis, bt, wt):
    return pl.pallas_call(
        _layer_body,
        grid=(_GRID,),
        in_specs=[_spec_half, _spec_half, _spec_col, _spec_b, _spec_w],
        out_specs=[_spec_row, _spec_half],
        out_shape=[
            jax.ShapeDtypeStruct((N, D), jnp.float32),
            jax.ShapeDtypeStruct((NQ, N, DQ), jnp.float32),
        ],
    )(p, g, dis, bt, wt)


def kernel(x, edge_index, edge_weight, W1, b1, W2, b2):
    src = edge_index[0].astype(jnp.int32)
    dst = edge_index[1].astype(jnp.int32)
    ew = edge_weight.astype(jnp.float32)
    pad = E_PAD - E
    pidx = jnp.arange(pad, dtype=jnp.int32) % N
    srcp = jnp.concatenate([src, pidx])
    dstp = jnp.concatenate([dst, pidx])
    ewp = jnp.concatenate([ew, jnp.zeros((pad,), jnp.float32)])
    srcr = srcp.reshape(NS, NB, BATCH)
    dstr = dstp.reshape(NS, NB, BATCH)
    ewf = ewp.reshape(NS, 1, T_EDGES)

    n_steps = 3 + jnp.minimum(jnp.min(edge_weight[:8]), 0.0).astype(jnp.int32)
    bs = jnp.stack([b1.reshape(1, D), b2.reshape(1, D)])
    ws = jnp.stack([W1, W2, W2])

    def cond(state):
        i, _, _, _ = state
        return i < n_steps

    def step(state):
        i, g, dis, _ = state
        flag = jnp.full((16,), (i == 0).astype(jnp.int32))
        a = _agg_call(g, srcr, dstr, ewf, flag)

        def first(_arg):
            d0 = lax.dynamic_slice(a, (0, 0, 0), (1, N, 1))[0]
            g1, dis1 = _lin1(d0, x, W1)
            return g1, dis1, x

        def later(_arg):
            y, g_next = _layer(
                a, g, dis,
                lax.dynamic_index_in_dim(bs, i - 1, keepdims=False),
                lax.dynamic_index_in_dim(ws, i, keepdims=False))
            return g_next, dis, y

        g_next, dis_next, y = lax.cond(i == 0, first, later, None)
        return i + 1, g_next, dis_next, y

    g_ones = jnp.ones((NQ, N, DQ), jnp.float32)
    _, _, _, out = lax.while_loop(
        cond, step, (jnp.int32(0), g_ones, jnp.zeros((N, 1), jnp.float32), x))
    return out


# speedup vs baseline: 1.0538x; 1.0200x over previous
import jax
import jax.numpy as jnp
from jax import lax
from jax.experimental import pallas as pl
from jax.experimental.pallas import tpu as pltpu
from jax.experimental.pallas import tpu_sc as plsc

N = 10000
D = 128
DQ = D // 4
NQ = 4
E = 320000
NC = 2
NS = 16
NW = NC * NS
BATCH = 128
NB_DEG = 80
NB = 160
T_EDGES = NB * BATCH
E_PAD = NS * T_EDGES
NROWS = N
ROWS_PER_TILE = NROWS // NS
RFULL = ROWS_PER_TILE // BATCH
RTAIL = ROWS_PER_TILE - RFULL * BATCH


def _agg_body(g, srcr, dstr, ewf, flagr, out, srcb, dstb, ewb, flagb,
              rows0, rows1, rows2, rows3, rows4, rows5, rows6, rows7,
              gs0, gs1, gs2, gs3, gs4, gs5, gs6, gs7,
              ss0, ss1, ss2, ss3, ss4, ss5, ss6, ss7, acc):
    c = lax.axis_index("c")
    s = lax.axis_index("s")
    rows = [rows0, rows1, rows2, rows3, rows4, rows5, rows6, rows7]
    gsem = [gs0, gs1, gs2, gs3, gs4, gs5, gs6, gs7]
    ssem = [ss0, ss1, ss2, ss3, ss4, ss5, ss6, ss7]

    pltpu.sync_copy(srcr.at[s], srcb)
    pltpu.sync_copy(dstr.at[s], dstb)
    pltpu.sync_copy(ewf.at[s, 0], ewb)
    pltpu.sync_copy(flagr, flagb)
    one_pass = flagb[...][0]

    def z_body(j, carry):
        for k in range(DQ // 16):
            rows0[j, pl.ds(k * 16, 16)] = jnp.zeros((16,), jnp.float32)
        return carry

    base = s * ROWS_PER_TILE

    def fire_gather(q, slot, b):
        pltpu.async_copy(g.at[q].at[srcb.at[b]], rows[slot], gsem[slot])

    def wait_gather(slot):
        pltpu.make_async_copy(g.at[0].at[pl.ds(0, BATCH)], rows[slot],
                              gsem[slot]).wait()

    def wait_scatter(slot):
        pltpu.make_async_copy(g.at[0].at[pl.ds(0, BATCH)], rows[slot],
                              ssem[slot]).wait()

    def run_pass(q):
        lax.fori_loop(0, BATCH, z_body, None)
        for i in range(RFULL):
            pltpu.sync_copy(rows0, acc.at[pl.ds(base + i * BATCH, BATCH)])
        pltpu.sync_copy(rows0.at[pl.ds(0, RTAIL)],
                        acc.at[pl.ds(base + RFULL * BATCH, RTAIL)])
        plsc.subcore_barrier()

        for pr in range(7):
            fire_gather(q, pr, pr)

        def grp(t, carry):
            for r in range(8):
                b = t * 8 + r
                wait_gather(r)
                eb = b * BATCH

                def sc_body(qq, carry2):
                    chunk = ewb[pl.ds(eb + qq * 16, 16)]
                    for t2 in range(16):
                        ev = chunk[t2]
                        j = qq * 16 + t2
                        for k in range(DQ // 16):
                            sl = pl.ds(k * 16, 16)
                            rows[r][j, sl] = rows[r][j, sl] * ev
                    return carry2

                lax.fori_loop(0, BATCH // 16, sc_body, None)
                pltpu.async_copy(rows[r], acc.at[dstb.at[b]], ssem[r],
                                 add=True)

                slot_n = (r + 7) % 8

                @pl.when(b + 7 < NB)
                def _():
                    @pl.when(b >= 1)
                    def _():
                        wait_scatter(slot_n)
                    fire_gather(q, slot_n, b + 7)
            return carry

        lax.fori_loop(0, NB // 8, grp, None)
        for r in range(8):
            wait_scatter(r)
        plsc.subcore_barrier()
        sl = pl.ds(base, ROWS_PER_TILE)
        pltpu.sync_copy(acc.at[sl], out.at[q, sl])
        plsc.subcore_barrier()

    def run_deg():
        def zr_body(j, carry):
            for rr in range(8):
                for k in range(DQ // 16):
                    rows[rr][j, pl.ds(k * 16, 16)] = jnp.zeros((16,),
                                                              jnp.float32)
            return carry

        lax.fori_loop(0, BATCH, zr_body, None)
        for i in range(RFULL):
            pltpu.sync_copy(rows0, acc.at[pl.ds(base + i * BATCH, BATCH)])
        pltpu.sync_copy(rows0.at[pl.ds(0, RTAIL)],
                        acc.at[pl.ds(base + RFULL * BATCH, RTAIL)])
        plsc.subcore_barrier()

        def dgrp(t, carry):
            for r in range(8):
                b = t * 8 + r
                eb = b * BATCH

                @pl.when(b >= 8)
                def _():
                    wait_scatter(r)

                def dfill(qq, carry2):
                    chunk = ewb[pl.ds(eb + qq * 16, 16)]
                    for t2 in range(16):
                        j = qq * 16 + t2
                        rows[r][j, pl.ds(0, 16)] = jnp.full(
                            (16,), chunk[t2], jnp.float32)
                    return carry2

                lax.fori_loop(0, BATCH // 16, dfill, None)
                pltpu.async_copy(rows[r], acc.at[dstb.at[b]], ssem[r],
                                 add=True)
            return carry

        lax.fori_loop(0, NB // 8, dgrp, None)
        for r in range(8):
            wait_scatter(r)
        plsc.subcore_barrier()
        sl = pl.ds(base, ROWS_PER_TILE)
        pltpu.sync_copy(acc.at[sl], out.at[0, sl])

    @pl.when(one_pass == 0)
    def _():
        run_pass(c)
        run_pass(2 + c)

    @pl.when(one_pass == 1)
    def _():
        run_deg()


_agg_call = pl.kernel(
    _agg_body,
    out_type=jax.ShapeDtypeStruct((NQ, NROWS, DQ), jnp.float32),
    compiler_params=pltpu.CompilerParams(use_tc_tiling_on_sc=False),
    mesh=plsc.VectorSubcoreMesh(core_axis_name="c", subcore_axis_name="s"),
    scratch_types=[
        pltpu.VMEM((NB, BATCH), jnp.int32),
        pltpu.VMEM((NB, BATCH), jnp.int32),
        pltpu.VMEM((T_EDGES,), jnp.float32),
        pltpu.VMEM((16,), jnp.int32),
        pltpu.VMEM((BATCH, DQ), jnp.float32),
        pltpu.VMEM((BATCH, DQ), jnp.float32),
        pltpu.VMEM((BATCH, DQ), jnp.float32),
        pltpu.VMEM((BATCH, DQ), jnp.float32),
        pltpu.VMEM((BATCH, DQ), jnp.float32),
        pltpu.VMEM((BATCH, DQ), jnp.float32),
        pltpu.VMEM((BATCH, DQ), jnp.float32),
        pltpu.VMEM((BATCH, DQ), jnp.float32),
        pltpu.SemaphoreType.DMA,
        pltpu.SemaphoreType.DMA,
        pltpu.SemaphoreType.DMA,
        pltpu.SemaphoreType.DMA,
        pltpu.SemaphoreType.DMA,
        pltpu.SemaphoreType.DMA,
        pltpu.SemaphoreType.DMA,
        pltpu.SemaphoreType.DMA,
        pltpu.SemaphoreType.DMA,
        pltpu.SemaphoreType.DMA,
        pltpu.SemaphoreType.DMA,
        pltpu.SemaphoreType.DMA,
        pltpu.SemaphoreType.DMA,
        pltpu.SemaphoreType.DMA,
        pltpu.SemaphoreType.DMA,
        pltpu.SemaphoreType.DMA,
        pltpu.VMEM_SHARED((NROWS, DQ), jnp.float32),
    ],
)


_BLK = 1000
_GRID = N // _BLK

_spec_col = pl.BlockSpec((_BLK, 1), lambda i: (i, 0))
_spec_row = pl.BlockSpec((_BLK, D), lambda i: (i, 0))
_spec_half = pl.BlockSpec((NQ, _BLK, DQ), lambda i: (0, i, 0))
_spec_w = pl.BlockSpec((D, D), lambda i: (0, 0))
_spec_b = pl.BlockSpec((1, D), lambda i: (0, 0))


def _lin1_body(d0, x, w1, g_out, dis_out):
    dis = lax.rsqrt(d0[...] + 1.0)
    h = jnp.dot(x[...], w1[...], preferred_element_type=jnp.float32) * dis
    for q in range(NQ):
        g_out[q] = h[:, q * DQ:(q + 1) * DQ]
    dis_out[...] = dis


def _lin1(d0, x, w1):
    return pl.pallas_call(
        _lin1_body,
        grid=(_GRID,),
        in_specs=[_spec_col, _spec_row, _spec_w],
        out_specs=[_spec_half, _spec_col],
        out_shape=[
            jax.ShapeDtypeStruct((NQ, N, DQ), jnp.float32),
            jax.ShapeDtypeStruct((N, 1), jnp.float32),
        ],
    )(d0, x, w1)


def _layer_body(p, g, dis, bt, wt, y_out, gn_out):
    dv = dis[...]
    y = jnp.concatenate([dv * (p[q] + g[q]) for q in range(NQ)],
                        axis=1) + bt[...]
    y_out[...] = y
    z = jnp.maximum(y, 0.0)
    h = jnp.dot(z, wt[...], preferred_element_type=jnp.float32) * dv
    for q in range(NQ):
        gn_out[q] = h[:, q * DQ:(q + 1) * DQ]


def _layer(p, g, dis, bt, wt):
    return pl.pallas_call(
        _layer_body,
        grid=(_GRID,),
        in_specs=[_spec_half, _spec_half, _spec_col, _spec_b, _spec_w],
        out_specs=[_spec_row, _spec_half],
        out_shape=[
            jax.ShapeDtypeStruct((N, D), jnp.float32),
            jax.ShapeDtypeStruct((NQ, N, DQ), jnp.float32),
        ],
    )(p, g, dis, bt, wt)


def kernel(x, edge_index, edge_weight, W1, b1, W2, b2):
    src = edge_index[0].astype(jnp.int32)
    dst = edge_index[1].astype(jnp.int32)
    ew = edge_weight.astype(jnp.float32)
    pad = E_PAD - E
    pidx = jnp.arange(pad, dtype=jnp.int32) % N
    srcp = jnp.concatenate([src, pidx])
    dstp = jnp.concatenate([dst, pidx])
    ewp = jnp.concatenate([ew, jnp.zeros((pad,), jnp.float32)])
    srcr = srcp.reshape(NS, NB, BATCH)
    dstr = dstp.reshape(NS, NB, BATCH)
    ewf = ewp.reshape(NS, 1, T_EDGES)

    n_steps = 3 + jnp.minimum(jnp.min(edge_weight[:8]), 0.0).astype(jnp.int32)
    bs = jnp.stack([b1.reshape(1, D), b2.reshape(1, D)])
    ws = jnp.stack([W1, W2, W2])

    def cond(state):
        i, _, _, _ = state
        return i < n_steps

    def step(state):
        i, g, dis, _ = state
        flag = jnp.full((16,), (i == 0).astype(jnp.int32))
        a = _agg_call(g, srcr, dstr, ewf, flag)

        def first(_arg):
            d0 = lax.dynamic_slice(a, (0, 0, 0), (1, N, 1))[0]
            g1, dis1 = _lin1(d0, x, W1)
            return g1, dis1, x

        def later(_arg):
            y, g_next = _layer(
                a, g, dis,
                lax.dynamic_index_in_dim(bs, i - 1, keepdims=False),
                lax.dynamic_index_in_dim(ws, i, keepdims=False))
            return g_next, dis, y

        g_next, dis_next, y = lax.cond(i == 0, first, later, None)
        return i + 1, g_next, dis_next, y

    g_ones = jnp.ones((NQ, N, DQ), jnp.float32)
    _, _, _, out = lax.while_loop(
        cond, step, (jnp.int32(0), g_ones, jnp.zeros((N, 1), jnp.float32), x))
    return out
